# baseline (speedup 1.0000x reference)
import functools

import numpy as np
import jax
import jax.numpy as jnp
from jax import lax
from jax.experimental import pallas as pl
from jax.experimental.pallas import tpu as pltpu

_ROT = ((13, 15, 26, 6), (17, 29, 16, 24))
_TINY = np.float32(np.finfo(np.float32).tiny)
_ONE_BITS = np.uint32(0x3F800000)
_P = np.float32(0.3)
_GROUP = 16


def _threefry_bits(ks0, ks1, ks2, x0, x1, inj=None):
    ks = (ks0, ks1, ks2)
    for i in range(5):
        for r in _ROT[i % 2]:
            x0 = x0 + x1
            x1 = (jnp.left_shift(x1, np.uint32(r))
                  | jnp.right_shift(x1, np.uint32(32 - r))) ^ x0
        x0 = x0 + ks[(i + 1) % 3]
        if inj is None:
            x1 = x1 + ks[(i + 2) % 3] + np.uint32(i + 1)
        else:
            x1 = x1 + inj[i]
    return x0 ^ x1


def _bits_to_unit_float(bits):
    fb = jnp.right_shift(bits, np.uint32(9)) | _ONE_BITS
    return lax.bitcast_convert_type(fb, jnp.float32) - np.float32(1.0)


def _augment_body(kd_ref, lt_ref, ls_ref, yf_ref, y_ref, o_ref, *,
                  L, n_classes, w, group):
    br, bw = y_ref.shape

    lane = (pl.program_id(1) * bw
            + lax.broadcasted_iota(jnp.int32, (br, bw), 1))
    seg = jnp.right_shift(lane * np.int32(41944), 8 + 15)

    yfv = jnp.zeros((br, bw), jnp.int32)
    for k in range(group):
        col = yf_ref[:, k][:, None]
        yfv = jnp.where(seg == np.int32(k), col, yfv)
    itv = yfv == np.int32(60)

    ka0, ka1 = kd_ref[0, 0], kd_ref[0, 1]
    kb0, kb1 = kd_ref[1, 0], kd_ref[1, 1]
    kc0, kc1 = kd_ref[2, 0], kd_ref[2, 1]

    row = (pl.program_id(0) * br
           + lax.broadcasted_iota(jnp.int32, (br, bw), 0))
    lin = row * np.int32(w) + lane

    ks0 = jnp.where(itv, ka0, kb0)
    ks1 = jnp.where(itv, ka1, kb1)
    ks2 = ks0 ^ ks1 ^ np.uint32(0x1BD11BDA)

    x1c0 = lax.bitcast_convert_type(lin * np.int32(n_classes),
                                    jnp.uint32) + ks1

    ksl = (ks0, ks1, ks2)
    inj = tuple(ksl[(i + 2) % 3] + np.uint32(i + 1) for i in range(5))

    def one_class(x1c, lt_c, ls_c, best, bidx, civ):
        bits = _threefry_bits(ks0, ks1, ks2, ks0, x1c, inj)
        f = _bits_to_unit_float(bits)
        uu = f + _TINY
        g = -jnp.log(-jnp.log(uu))
        logit = jnp.where(itv, lt_c, ls_c)
        val = g + logit
        upd = val > best
        best = jnp.where(upd, val, best)
        bidx = jnp.where(upd, civ, bidx)
        return best, bidx

    UNROLL = 4
    def cls_body(c, carry):
        best, bidx, x1c, civ = carry
        c0 = UNROLL * c
        for t in range(UNROLL):
            best, bidx = one_class(x1c + np.uint32(t),
                                   lt_ref[c0 + t], ls_ref[c0 + t],
                                   best, bidx, civ + np.int32(t))
        return (best, bidx, x1c + np.uint32(UNROLL),
                civ + np.int32(UNROLL))

    init = (jnp.full((br, bw), -np.inf, jnp.float32),
            jnp.zeros((br, bw), jnp.int32),
            x1c0,
            jnp.zeros((br, bw), jnp.int32))
    best, bidx, x1c, civ = lax.fori_loop(0, n_classes // UNROLL,
                                         cls_body, init)
    for t in range(n_classes % UNROLL):
        best, bidx = one_class(x1c + np.uint32(t),
                               lt_ref[n_classes - n_classes % UNROLL + t],
                               ls_ref[n_classes - n_classes % UNROLL + t],
                               best, bidx, civ + np.int32(t))

    kc2 = kc0 ^ kc1 ^ np.uint32(0x1BD11BDA)
    lin_u = lax.bitcast_convert_type(lin, jnp.uint32)
    u = _bits_to_unit_float(_threefry_bits(kc0, kc1, kc2, kc0, lin_u + kc1))
    y = y_ref[...]
    noise_mask = (u < _P) & (y < np.int32(59))

    o_ref[...] = jnp.where(noise_mask, bidx, y)


def kernel(y, train_counts, supp_counts):
    B, L = y.shape
    n_classes = train_counts.shape[0]
    fr = B // _GROUP
    w = _GROUP * L
    br = 32 if fr % 32 == 0 else 1
    bw = 640 if w % 640 == 0 else w

    y32 = y.astype(jnp.int32)
    yf = y32[:, 0].reshape(fr, _GROUP)
    y32 = y32.reshape(fr, w)
    kd = jax.random.key_data(jax.random.split(jax.random.key(42), 3))
    kd = kd.astype(jnp.uint32)
    lt = jnp.log(train_counts.astype(jnp.float32))
    ls = jnp.log(supp_counts.astype(jnp.float32))

    with jax.enable_x64(False):
        out = pl.pallas_call(
            functools.partial(_augment_body, L=L, n_classes=n_classes,
                              w=w, group=_GROUP),
            grid=(fr // br, w // bw),
            in_specs=[
                pl.BlockSpec(memory_space=pltpu.SMEM),
                pl.BlockSpec(memory_space=pltpu.SMEM),
                pl.BlockSpec(memory_space=pltpu.SMEM),
                pl.BlockSpec((br, _GROUP), lambda i, j: (i, 0)),
                pl.BlockSpec((br, bw), lambda i, j: (i, j)),
            ],
            out_specs=pl.BlockSpec((br, bw), lambda i, j: (i, j)),
            out_shape=jax.ShapeDtypeStruct((fr, w), jnp.int32),
            compiler_params=pltpu.CompilerParams(
                dimension_semantics=("parallel", "parallel")),
        )(kd, lt, ls, yf, y32)

    return out.reshape(B, L).astype(y.dtype)


# speedup vs baseline: 1.0706x; 1.0706x over previous
import functools

import numpy as np
import jax
import jax.numpy as jnp
from jax import lax
from jax.experimental import pallas as pl
from jax.experimental.pallas import tpu as pltpu

_ROT = ((13, 15, 26, 6), (17, 29, 16, 24))
_TINY = np.float32(np.finfo(np.float32).tiny)
_ONE_BITS = np.uint32(0x3F800000)
_P = np.float32(0.3)
_GROUP = 16


def _threefry_bits(ks0, ks1, ks2, x0, x1, inj=None):
    ks = (ks0, ks1, ks2)
    for i in range(5):
        for r in _ROT[i % 2]:
            x0 = x0 + x1
            x1 = (jnp.left_shift(x1, np.uint32(r))
                  | jnp.right_shift(x1, np.uint32(32 - r))) ^ x0
        x0 = x0 + ks[(i + 1) % 3]
        if inj is None:
            x1 = x1 + ks[(i + 2) % 3] + np.uint32(i + 1)
        else:
            x1 = x1 + inj[i]
    return x0 ^ x1


def _bits_to_unit_float(bits):
    fb = jnp.right_shift(bits, np.uint32(9)) | _ONE_BITS
    return lax.bitcast_convert_type(fb, jnp.float32) - np.float32(1.0)


def _augment_body(kd_ref, lt_ref, ls_ref, yf_ref, y_ref, o_ref, *,
                  L, n_classes, w, group):
    br, bw = y_ref.shape

    lane = (pl.program_id(1) * bw
            + lax.broadcasted_iota(jnp.int32, (br, bw), 1))
    seg = jnp.right_shift(lane * np.int32(41944), 8 + 15)

    yfv = jnp.zeros((br, bw), jnp.int32)
    for k in range(group):
        col = yf_ref[:, k][:, None]
        yfv = jnp.where(seg == np.int32(k), col, yfv)
    itv = yfv == np.int32(60)

    ka0, ka1 = kd_ref[0, 0], kd_ref[0, 1]
    kb0, kb1 = kd_ref[1, 0], kd_ref[1, 1]
    kc0, kc1 = kd_ref[2, 0], kd_ref[2, 1]

    row = (pl.program_id(0) * br
           + lax.broadcasted_iota(jnp.int32, (br, bw), 0))
    lin = row * np.int32(w) + lane

    ks0 = jnp.where(itv, ka0, kb0)
    ks1 = jnp.where(itv, ka1, kb1)
    ks2 = ks0 ^ ks1 ^ np.uint32(0x1BD11BDA)

    x1c0 = lax.bitcast_convert_type(lin * np.int32(n_classes),
                                    jnp.uint32) + ks1

    ksl = (ks0, ks1, ks2)
    inj = tuple(ksl[(i + 2) % 3] + np.uint32(i + 1) for i in range(5))

    def one_class(x1c, lt_c, ls_c, best, bidx, civ):
        bits = _threefry_bits(ks0, ks1, ks2, ks0, x1c, inj)
        f = _bits_to_unit_float(bits)
        uu = f + _TINY
        g = -jnp.log(-jnp.log(uu))
        logit = jnp.where(itv, lt_c, ls_c)
        val = g + logit
        upd = val > best
        best = jnp.where(upd, val, best)
        bidx = jnp.where(upd, civ, bidx)
        return best, bidx

    UNROLL = 8
    def cls_body(c, carry):
        best, bidx, x1c, civ = carry
        c0 = UNROLL * c
        for t in range(UNROLL):
            best, bidx = one_class(x1c + np.uint32(t),
                                   lt_ref[c0 + t], ls_ref[c0 + t],
                                   best, bidx, civ + np.int32(t))
        return (best, bidx, x1c + np.uint32(UNROLL),
                civ + np.int32(UNROLL))

    init = (jnp.full((br, bw), -np.inf, jnp.float32),
            jnp.zeros((br, bw), jnp.int32),
            x1c0,
            jnp.zeros((br, bw), jnp.int32))
    best, bidx, x1c, civ = lax.fori_loop(0, n_classes // UNROLL,
                                         cls_body, init)
    for t in range(n_classes % UNROLL):
        best, bidx = one_class(x1c + np.uint32(t),
                               lt_ref[n_classes - n_classes % UNROLL + t],
                               ls_ref[n_classes - n_classes % UNROLL + t],
                               best, bidx, civ + np.int32(t))

    kc2 = kc0 ^ kc1 ^ np.uint32(0x1BD11BDA)
    lin_u = lax.bitcast_convert_type(lin, jnp.uint32)
    u = _bits_to_unit_float(_threefry_bits(kc0, kc1, kc2, kc0, lin_u + kc1))
    y = y_ref[...]
    noise_mask = (u < _P) & (y < np.int32(59))

    o_ref[...] = jnp.where(noise_mask, bidx, y)


def kernel(y, train_counts, supp_counts):
    B, L = y.shape
    n_classes = train_counts.shape[0]
    fr = B // _GROUP
    w = _GROUP * L
    br = 16 if fr % 16 == 0 else 1
    bw = 640 if w % 640 == 0 else w

    y32 = y.astype(jnp.int32)
    yf = y32[:, 0].reshape(fr, _GROUP)
    y32 = y32.reshape(fr, w)
    kd = jax.random.key_data(jax.random.split(jax.random.key(42), 3))
    kd = kd.astype(jnp.uint32)
    lt = jnp.log(train_counts.astype(jnp.float32))
    ls = jnp.log(supp_counts.astype(jnp.float32))

    with jax.enable_x64(False):
        out = pl.pallas_call(
            functools.partial(_augment_body, L=L, n_classes=n_classes,
                              w=w, group=_GROUP),
            grid=(fr // br, w // bw),
            in_specs=[
                pl.BlockSpec(memory_space=pltpu.SMEM),
                pl.BlockSpec(memory_space=pltpu.SMEM),
                pl.BlockSpec(memory_space=pltpu.SMEM),
                pl.BlockSpec((br, _GROUP), lambda i, j: (i, 0)),
                pl.BlockSpec((br, bw), lambda i, j: (i, j)),
            ],
            out_specs=pl.BlockSpec((br, bw), lambda i, j: (i, j)),
            out_shape=jax.ShapeDtypeStruct((fr, w), jnp.int32),
            compiler_params=pltpu.CompilerParams(
                dimension_semantics=("parallel", "parallel")),
        )(kd, lt, ls, yf, y32)

    return out.reshape(B, L).astype(y.dtype)


# speedup vs baseline: 1.0769x; 1.0059x over previous
import functools

import numpy as np
import jax
import jax.numpy as jnp
from jax import lax
from jax.experimental import pallas as pl
from jax.experimental.pallas import tpu as pltpu

_ROT = ((13, 15, 26, 6), (17, 29, 16, 24))
_TINY = np.float32(np.finfo(np.float32).tiny)
_ONE_BITS = np.uint32(0x3F800000)
_P = np.float32(0.3)
_GROUP = 16


def _threefry_bits(ks0, ks1, ks2, x0, x1, inj=None):
    ks = (ks0, ks1, ks2)
    for i in range(5):
        for r in _ROT[i % 2]:
            x0 = x0 + x1
            x1 = (jnp.left_shift(x1, np.uint32(r))
                  | jnp.right_shift(x1, np.uint32(32 - r))) ^ x0
        x0 = x0 + ks[(i + 1) % 3]
        if inj is None:
            x1 = x1 + ks[(i + 2) % 3] + np.uint32(i + 1)
        else:
            x1 = x1 + inj[i]
    return x0 ^ x1


def _bits_to_unit_float(bits):
    fb = jnp.right_shift(bits, np.uint32(9)) | _ONE_BITS
    return lax.bitcast_convert_type(fb, jnp.float32) - np.float32(1.0)


def _augment_body(kd_ref, lt_ref, ls_ref, yf_ref, y_ref, o_ref, *,
                  L, n_classes, w, group):
    br, bw = y_ref.shape

    lane = (pl.program_id(1) * bw
            + lax.broadcasted_iota(jnp.int32, (br, bw), 1))
    seg = jnp.right_shift(lane * np.int32(41944), 8 + 15)

    yfv = jnp.zeros((br, bw), jnp.int32)
    for k in range(group):
        col = yf_ref[:, k][:, None]
        yfv = jnp.where(seg == np.int32(k), col, yfv)
    itv = yfv == np.int32(60)

    ka0, ka1 = kd_ref[0, 0], kd_ref[0, 1]
    kb0, kb1 = kd_ref[1, 0], kd_ref[1, 1]
    kc0, kc1 = kd_ref[2, 0], kd_ref[2, 1]

    row = (pl.program_id(0) * br
           + lax.broadcasted_iota(jnp.int32, (br, bw), 0))
    lin = row * np.int32(w) + lane

    ks0 = jnp.where(itv, ka0, kb0)
    ks1 = jnp.where(itv, ka1, kb1)
    ks2 = ks0 ^ ks1 ^ np.uint32(0x1BD11BDA)

    x1c0 = lax.bitcast_convert_type(lin * np.int32(n_classes),
                                    jnp.uint32) + ks1

    ksl = (ks0, ks1, ks2)
    inj = tuple(ksl[(i + 2) % 3] + np.uint32(i + 1) for i in range(5))

    def one_class(x1c, lt_c, ls_c, best, bidx, civ):
        bits = _threefry_bits(ks0, ks1, ks2, ks0, x1c, inj)
        f = _bits_to_unit_float(bits)
        uu = f + _TINY
        g = -jnp.log(-jnp.log(uu))
        logit = jnp.where(itv, lt_c, ls_c)
        val = g + logit
        upd = val > best
        best = jnp.where(upd, val, best)
        bidx = jnp.where(upd, civ, bidx)
        return best, bidx

    UNROLL = 16
    def cls_body(c, carry):
        best, bidx, x1c, civ = carry
        c0 = UNROLL * c
        for t in range(UNROLL):
            best, bidx = one_class(x1c + np.uint32(t),
                                   lt_ref[c0 + t], ls_ref[c0 + t],
                                   best, bidx, civ + np.int32(t))
        return (best, bidx, x1c + np.uint32(UNROLL),
                civ + np.int32(UNROLL))

    init = (jnp.full((br, bw), -np.inf, jnp.float32),
            jnp.zeros((br, bw), jnp.int32),
            x1c0,
            jnp.zeros((br, bw), jnp.int32))
    best, bidx, x1c, civ = lax.fori_loop(0, n_classes // UNROLL,
                                         cls_body, init)
    for t in range(n_classes % UNROLL):
        best, bidx = one_class(x1c + np.uint32(t),
                               lt_ref[n_classes - n_classes % UNROLL + t],
                               ls_ref[n_classes - n_classes % UNROLL + t],
                               best, bidx, civ + np.int32(t))

    kc2 = kc0 ^ kc1 ^ np.uint32(0x1BD11BDA)
    lin_u = lax.bitcast_convert_type(lin, jnp.uint32)
    u = _bits_to_unit_float(_threefry_bits(kc0, kc1, kc2, kc0, lin_u + kc1))
    y = y_ref[...]
    noise_mask = (u < _P) & (y < np.int32(59))

    o_ref[...] = jnp.where(noise_mask, bidx, y)


def kernel(y, train_counts, supp_counts):
    B, L = y.shape
    n_classes = train_counts.shape[0]
    fr = B // _GROUP
    w = _GROUP * L
    br = 16 if fr % 16 == 0 else 1
    bw = 640 if w % 640 == 0 else w

    y32 = y.astype(jnp.int32)
    yf = y32[:, 0].reshape(fr, _GROUP)
    y32 = y32.reshape(fr, w)
    kd = jax.random.key_data(jax.random.split(jax.random.key(42), 3))
    kd = kd.astype(jnp.uint32)
    lt = jnp.log(train_counts.astype(jnp.float32))
    ls = jnp.log(supp_counts.astype(jnp.float32))

    with jax.enable_x64(False):
        out = pl.pallas_call(
            functools.partial(_augment_body, L=L, n_classes=n_classes,
                              w=w, group=_GROUP),
            grid=(fr // br, w // bw),
            in_specs=[
                pl.BlockSpec(memory_space=pltpu.SMEM),
                pl.BlockSpec(memory_space=pltpu.SMEM),
                pl.BlockSpec(memory_space=pltpu.SMEM),
                pl.BlockSpec((br, _GROUP), lambda i, j: (i, 0)),
                pl.BlockSpec((br, bw), lambda i, j: (i, j)),
            ],
            out_specs=pl.BlockSpec((br, bw), lambda i, j: (i, j)),
            out_shape=jax.ShapeDtypeStruct((fr, w), jnp.int32),
            compiler_params=pltpu.CompilerParams(
                dimension_semantics=("parallel", "parallel")),
        )(kd, lt, ls, yf, y32)

    return out.reshape(B, L).astype(y.dtype)


# speedup vs baseline: 1.0819x; 1.0047x over previous
import functools

import numpy as np
import jax
import jax.numpy as jnp
from jax import lax
from jax.experimental import pallas as pl
from jax.experimental.pallas import tpu as pltpu

_ROT = ((13, 15, 26, 6), (17, 29, 16, 24))
_TINY = np.float32(np.finfo(np.float32).tiny)
_ONE_BITS = np.uint32(0x3F800000)
_P = np.float32(0.3)
_GROUP = 16


def _threefry_bits(ks0, ks1, ks2, x0, x1, inj=None):
    ks = (ks0, ks1, ks2)
    for i in range(5):
        for r in _ROT[i % 2]:
            x0 = x0 + x1
            x1 = (jnp.left_shift(x1, np.uint32(r))
                  | jnp.right_shift(x1, np.uint32(32 - r))) ^ x0
        x0 = x0 + ks[(i + 1) % 3]
        if inj is None:
            x1 = x1 + ks[(i + 2) % 3] + np.uint32(i + 1)
        else:
            x1 = x1 + inj[i]
    return x0 ^ x1


def _bits_to_unit_float(bits):
    fb = jnp.right_shift(bits, np.uint32(9)) | _ONE_BITS
    return lax.bitcast_convert_type(fb, jnp.float32) - np.float32(1.0)


def _augment_body(kd_ref, lt_ref, ls_ref, yf_ref, y_ref, o_ref, *,
                  L, n_classes, w, group):
    br, bw = y_ref.shape

    lane = (pl.program_id(1) * bw
            + lax.broadcasted_iota(jnp.int32, (br, bw), 1))
    seg = jnp.right_shift(lane * np.int32(41944), 8 + 15)

    yfv = jnp.zeros((br, bw), jnp.int32)
    for k in range(group):
        col = yf_ref[:, k][:, None]
        yfv = jnp.where(seg == np.int32(k), col, yfv)
    itv = yfv == np.int32(60)

    ka0, ka1 = kd_ref[0, 0], kd_ref[0, 1]
    kb0, kb1 = kd_ref[1, 0], kd_ref[1, 1]
    kc0, kc1 = kd_ref[2, 0], kd_ref[2, 1]

    row = (pl.program_id(0) * br
           + lax.broadcasted_iota(jnp.int32, (br, bw), 0))
    lin = row * np.int32(w) + lane

    ks0 = jnp.where(itv, ka0, kb0)
    ks1 = jnp.where(itv, ka1, kb1)
    ks2 = ks0 ^ ks1 ^ np.uint32(0x1BD11BDA)

    x1c0 = lax.bitcast_convert_type(lin * np.int32(n_classes),
                                    jnp.uint32) + ks1

    ksl = (ks0, ks1, ks2)
    inj = tuple(ksl[(i + 2) % 3] + np.uint32(i + 1) for i in range(5))

    def one_class(x1c, lt_c, ls_c, best, bidx, civ):
        bits = _threefry_bits(ks0, ks1, ks2, ks0, x1c, inj)
        f = _bits_to_unit_float(bits)
        uu = f + _TINY
        g = -jnp.log(-jnp.log(uu))
        logit = jnp.where(itv, lt_c, ls_c)
        val = g + logit
        upd = val > best
        best = jnp.where(upd, val, best)
        bidx = jnp.where(upd, civ, bidx)
        return best, bidx

    UNROLL = 59
    def cls_body(c, carry):
        best, bidx, x1c, civ = carry
        c0 = UNROLL * c
        for t in range(UNROLL):
            best, bidx = one_class(x1c + np.uint32(t),
                                   lt_ref[c0 + t], ls_ref[c0 + t],
                                   best, bidx, civ + np.int32(t))
        return (best, bidx, x1c + np.uint32(UNROLL),
                civ + np.int32(UNROLL))

    init = (jnp.full((br, bw), -np.inf, jnp.float32),
            jnp.zeros((br, bw), jnp.int32),
            x1c0,
            jnp.zeros((br, bw), jnp.int32))
    best, bidx, x1c, civ = lax.fori_loop(0, n_classes // UNROLL,
                                         cls_body, init)
    for t in range(n_classes % UNROLL):
        best, bidx = one_class(x1c + np.uint32(t),
                               lt_ref[n_classes - n_classes % UNROLL + t],
                               ls_ref[n_classes - n_classes % UNROLL + t],
                               best, bidx, civ + np.int32(t))

    kc2 = kc0 ^ kc1 ^ np.uint32(0x1BD11BDA)
    lin_u = lax.bitcast_convert_type(lin, jnp.uint32)
    u = _bits_to_unit_float(_threefry_bits(kc0, kc1, kc2, kc0, lin_u + kc1))
    y = y_ref[...]
    noise_mask = (u < _P) & (y < np.int32(59))

    o_ref[...] = jnp.where(noise_mask, bidx, y)


def kernel(y, train_counts, supp_counts):
    B, L = y.shape
    n_classes = train_counts.shape[0]
    fr = B // _GROUP
    w = _GROUP * L
    br = 16 if fr % 16 == 0 else 1
    bw = 640 if w % 640 == 0 else w

    y32 = y.astype(jnp.int32)
    yf = y32[:, 0].reshape(fr, _GROUP)
    y32 = y32.reshape(fr, w)
    kd = jax.random.key_data(jax.random.split(jax.random.key(42), 3))
    kd = kd.astype(jnp.uint32)
    lt = jnp.log(train_counts.astype(jnp.float32))
    ls = jnp.log(supp_counts.astype(jnp.float32))

    with jax.enable_x64(False):
        out = pl.pallas_call(
            functools.partial(_augment_body, L=L, n_classes=n_classes,
                              w=w, group=_GROUP),
            grid=(fr // br, w // bw),
            in_specs=[
                pl.BlockSpec(memory_space=pltpu.SMEM),
                pl.BlockSpec(memory_space=pltpu.SMEM),
                pl.BlockSpec(memory_space=pltpu.SMEM),
                pl.BlockSpec((br, _GROUP), lambda i, j: (i, 0)),
                pl.BlockSpec((br, bw), lambda i, j: (i, j)),
            ],
            out_specs=pl.BlockSpec((br, bw), lambda i, j: (i, j)),
            out_shape=jax.ShapeDtypeStruct((fr, w), jnp.int32),
            compiler_params=pltpu.CompilerParams(
                dimension_semantics=("parallel", "parallel")),
        )(kd, lt, ls, yf, y32)

    return out.reshape(B, L).astype(y.dtype)
